# Initial kernel scaffold; baseline (speedup 1.0000x reference)
import jax
import jax.numpy as jnp
from jax.experimental import pallas as pl


def kernel(hidden_states, Wg, e_bias, Wu, Wd, Wu_s, Wd_s):
    raise NotImplementedError("write your pallas kernel here")


# speedup vs baseline: 1.8437x; 1.8437x over previous
import functools

import jax
import jax.numpy as jnp
import numpy as np
from jax.experimental import pallas as pl
from jax.experimental.pallas import tpu as pltpu

H = 1024
E = 16
I = 512
IS = 2048
N_GROUP = 4
GROUP_SIZE = E // N_GROUP
TOPK_GROUP = 2
TOP_K = 8
ROUTED_SCALE = 2.5

T = 2048
TBLK = 256


def _rank_desc(v):
    B, N = v.shape
    idx = jax.lax.broadcasted_iota(jnp.int32, (B, N), 1)
    rank = jnp.zeros((B, N), jnp.float32)
    for j in range(N):
        colv = v[:, j:j + 1]
        beats = jnp.logical_or(colv > v,
                               jnp.logical_and(colv == v, j < idx))
        rank = rank + jnp.where(beats, 1.0, 0.0)
    return rank


def _router_kernel(s_ref, bias_ref, cmb_ref):
    s = s_ref[...]
    sc = s + bias_ref[...]

    col = [sc[:, j:j + 1] for j in range(E)]

    top2 = []
    for g in range(N_GROUP):
        a, b, c, d = col[4 * g], col[4 * g + 1], col[4 * g + 2], col[4 * g + 3]
        hi1, lo1 = jnp.maximum(a, b), jnp.minimum(a, b)
        hi2, lo2 = jnp.maximum(c, d), jnp.minimum(c, d)
        top2.append(jnp.maximum(jnp.maximum(hi1 + hi2, hi1 + lo1), hi2 + lo2))

    lane = jax.lax.broadcasted_iota(jnp.int32, (TBLK, E), 1)
    zero = jnp.zeros((TBLK, E), jnp.float32)
    esel = zero
    for g in range(N_GROUP):
        grank = 0
        for g2 in range(N_GROUP):
            if g2 == g:
                continue
            beats = jnp.logical_or(
                top2[g2] > top2[g],
                jnp.logical_and(top2[g2] == top2[g], g2 < g))
            grank = grank + jnp.where(beats, 1, 0)
        gsel = grank < TOPK_GROUP
        gmask = jnp.logical_and(lane >= 4 * g, lane < 4 * (g + 1))
        esel = esel + jnp.where(jnp.logical_and(gsel, gmask), 1.0, 0.0)

    scores_for_choice = jnp.where(esel > 0.5, sc, 0.0)

    erank = _rank_desc(scores_for_choice)
    sel = erank < TOP_K

    tw = jnp.where(sel, s, 0.0)
    denom = jnp.sum(tw, axis=1, keepdims=True) + 1e-20
    cmb_ref[...] = tw * (ROUTED_SCALE / denom)


def _moe_kernel(cmb_ref, x_ref, wu_ref, wd_ref, wus_ref, wds_ref, out_ref):
    i = pl.program_id(0)
    routed = i < E
    ci = jnp.where(routed, i, E - 1)

    x = x_ref[...]
    wu = jnp.where(routed, wu_ref[0], wus_ref[...])
    wd = jnp.where(routed, wd_ref[0], wds_ref[...])

    h = jnp.dot(x, wu, preferred_element_type=jnp.float32)
    h = jnp.square(jnp.maximum(h, 0.0))
    y = jnp.dot(h, wd, preferred_element_type=jnp.float32)

    lane = jax.lax.broadcasted_iota(jnp.int32, (T, E), 1)
    w = jnp.sum(jnp.where(lane == ci, cmb_ref[...], 0.0), axis=1, keepdims=True)
    w = jnp.where(routed, w, 1.0)

    @pl.when(i == 0)
    def _init():
        out_ref[...] = y * w

    @pl.when(i > 0)
    def _acc():
        out_ref[...] += y * w


def _build(interpret=False):
    router = pl.pallas_call(
        _router_kernel,
        grid=(T // TBLK,),
        in_specs=[
            pl.BlockSpec((TBLK, E), lambda t: (t, 0)),
            pl.BlockSpec((1, E), lambda t: (0, 0)),
        ],
        out_specs=pl.BlockSpec((TBLK, E), lambda t: (t, 0)),
        out_shape=jax.ShapeDtypeStruct((T, E), jnp.float32),
        interpret=interpret,
    )

    nsteps = E + IS // I
    moe = pl.pallas_call(
        _moe_kernel,
        grid=(nsteps,),
        in_specs=[
            pl.BlockSpec((T, E), lambda i: (0, 0)),
            pl.BlockSpec((T, H), lambda i: (0, 0)),
            pl.BlockSpec((1, H, I), lambda i: (jnp.where(i < E, i, E - 1), 0, 0)),
            pl.BlockSpec((1, I, H), lambda i: (jnp.where(i < E, i, E - 1), 0, 0)),
            pl.BlockSpec((H, I), lambda i: (0, jnp.where(i < E, 0, i - E))),
            pl.BlockSpec((I, H), lambda i: (jnp.where(i < E, 0, i - E), 0)),
        ],
        out_specs=pl.BlockSpec((T, H), lambda i: (0, 0)),
        out_shape=jax.ShapeDtypeStruct((T, H), jnp.float32),
        compiler_params=pltpu.CompilerParams(
            dimension_semantics=("arbitrary",),
        ),
        interpret=interpret,
    )
    return router, moe


@functools.partial(jax.jit, static_argnames=("interpret",))
def _run(hidden_states, Wg, e_bias, Wu, Wd, Wu_s, Wd_s, interpret=False):
    router, moe = _build(interpret)
    x = hidden_states.reshape(T, H)
    s = jax.nn.sigmoid(x.astype(jnp.float32) @ Wg.T)
    cmb = router(s, e_bias.reshape(1, E))
    out = moe(cmb, x, Wu, Wd, Wu_s, Wd_s)
    return out.reshape(hidden_states.shape)


def kernel(hidden_states, Wg, e_bias, Wu, Wd, Wu_s, Wd_s):
    return _run(hidden_states, Wg, e_bias, Wu, Wd, Wu_s, Wd_s)
